# baseline (speedup 1.0000x reference)
import functools

import jax
import jax.numpy as jnp
from jax import lax
from jax.experimental import pallas as pl
from jax.experimental.pallas import tpu as pltpu
from jax.experimental.pallas import tpu_sc as plsc

NFEAT = 1000000
NEMB = 16
BATCH = 16384
NFIELDS = 26
NC, NS, NLANE = 2, 16, 16
NW = NC * NS
ROWS_PER_W = BATCH // NW
RCHUNK = 64
NCHUNK = ROWS_PER_W // RCHUNK

_mesh = plsc.VectorSubcoreMesh(core_axis_name="c", subcore_axis_name="s")


@functools.partial(
    pl.kernel,
    mesh=_mesh,
    compiler_params=pltpu.CompilerParams(use_tc_tiling_on_sc=False),
    out_type=jax.ShapeDtypeStruct((BATCH, NFIELDS, NEMB), jnp.float32),
    scratch_types=[
        pltpu.VMEM((2, RCHUNK, NFIELDS), jnp.int32),
        pltpu.VMEM((2, RCHUNK, NFIELDS), jnp.float32),
        pltpu.VMEM((2, RCHUNK, NFIELDS, NEMB), jnp.float32),
        pltpu.SemaphoreType.DMA,
        pltpu.SemaphoreType.DMA,
    ],
)
def _emb_lookup(w_hbm, idx_hbm, val_hbm, out_hbm, idx_v, val_v, rows_v, gsem, osem):
    wid = lax.axis_index("s") * NC + lax.axis_index("c")
    row0 = wid * ROWS_PER_W

    def stage(c, buf):
        b0 = row0 + c * RCHUNK
        pltpu.sync_copy(idx_hbm.at[pl.ds(b0, RCHUNK)], idx_v.at[buf])
        pltpu.sync_copy(val_hbm.at[pl.ds(b0, RCHUNK)], val_v.at[buf])

        def fire(r, carry):
            pltpu.async_copy(
                w_hbm.at[idx_v.at[buf, r]], rows_v.at[buf, r], gsem
            )
            return carry

        lax.fori_loop(0, RCHUNK, fire, 0)
        return pltpu.make_async_copy(
            out_hbm.at[pl.ds(b0, RCHUNK)], rows_v.at[buf], gsem
        )

    def scale(buf):
        def row_body(r, carry):
            va = val_v[buf, r, pl.ds(0, NLANE)]
            vb = val_v[buf, r, pl.ds(NFIELDS - NLANE, NLANE)]
            for f in range(NFIELDS):
                s = va[f] if f < NLANE else vb[f - (NFIELDS - NLANE)]
                rows_v[buf, r, f, :] = rows_v[buf, r, f, :] * s
            return carry

        lax.fori_loop(0, RCHUNK, row_body, 0)

    gathers = [None, None]
    out_copies = [None, None]
    gathers[0] = stage(0, 0)
    for c in range(NCHUNK):
        buf = c & 1
        if c + 1 < NCHUNK:
            if out_copies[1 - buf] is not None:
                out_copies[1 - buf].wait()
                out_copies[1 - buf] = None
            gathers[1 - buf] = stage(c + 1, 1 - buf)
        gathers[buf].wait()
        scale(buf)
        out_copies[buf] = pltpu.async_copy(
            rows_v.at[buf], out_hbm.at[pl.ds(row0 + c * RCHUNK, RCHUNK)], osem
        )
    for cp in out_copies:
        if cp is not None:
            cp.wait()


TCOLS = 16384
TGRID = -(-NFEAT // TCOLS)


def _transpose_body(wt_ref, out_ref):
    z3 = wt_ref[...].T.reshape(TCOLS // 8, 8, NEMB)
    out_ref[...] = jnp.concatenate(
        [z3[:, j, :] for j in range(8)], axis=1
    )


_w_to_rows = pl.pallas_call(
    _transpose_body,
    grid=(TGRID,),
    in_specs=[pl.BlockSpec((NEMB, TCOLS), lambda g: (0, g))],
    out_specs=pl.BlockSpec((TCOLS // 8, 128), lambda g: (g, 0)),
    out_shape=jax.ShapeDtypeStruct((NFEAT // 8, 128), jnp.float32),
)


def kernel(id, value, W):
    w_rows = _w_to_rows(W.T).reshape(NFEAT, NEMB)
    return _emb_lookup(w_rows, id.astype(jnp.int32), value)


# speedup vs baseline: 1.2625x; 1.2625x over previous
import functools

import jax
import jax.numpy as jnp
from jax import lax
from jax.experimental import pallas as pl
from jax.experimental.pallas import tpu as pltpu
from jax.experimental.pallas import tpu_sc as plsc

NFEAT = 1000000
NEMB = 16
BATCH = 16384
NFIELDS = 26
NC, NS, NLANE = 2, 16, 16
NW = NC * NS
ROWS_PER_W = BATCH // NW
RCHUNK = 64
NCHUNK = ROWS_PER_W // RCHUNK

_mesh = plsc.VectorSubcoreMesh(core_axis_name="c", subcore_axis_name="s")


@functools.partial(
    pl.kernel,
    mesh=_mesh,
    compiler_params=pltpu.CompilerParams(use_tc_tiling_on_sc=False),
    out_type=jax.ShapeDtypeStruct((BATCH, 32, 128), jnp.float32),
    scratch_types=[
        pltpu.VMEM((2, RCHUNK, NFIELDS), jnp.int32),
        pltpu.VMEM((2, RCHUNK, NFIELDS), jnp.float32),
        pltpu.VMEM((2, RCHUNK, 32, NEMB), jnp.float32),
        pltpu.SemaphoreType.DMA,
        pltpu.SemaphoreType.DMA,
    ],
)
def _emb_lookup(w_hbm, idx_hbm, val_hbm, out_hbm, idx_v, val_v, rows_v, gsem, osem):
    wid = lax.axis_index("s") * NC + lax.axis_index("c")
    row0 = wid * ROWS_PER_W

    def stage(c, buf):
        b0 = row0 + c * RCHUNK
        pltpu.sync_copy(idx_hbm.at[pl.ds(b0, RCHUNK)], idx_v.at[buf])
        pltpu.sync_copy(val_hbm.at[pl.ds(b0, RCHUNK)], val_v.at[buf])

        def fire(r, carry):
            pltpu.async_copy(
                w_hbm.at[idx_v.at[buf, r]],
                rows_v.at[buf, r, pl.ds(0, NFIELDS)],
                gsem,
            )
            return carry

        lax.fori_loop(0, RCHUNK, fire, 0)
        return pltpu.make_async_copy(
            out_hbm.at[pl.ds(b0, RCHUNK), pl.ds(0, NFIELDS), pl.ds(0, NEMB)],
            rows_v.at[buf, :, pl.ds(0, NFIELDS)],
            gsem,
        )

    def scale(buf):
        def row_body(r, carry):
            va = val_v[buf, r, pl.ds(0, NLANE)]
            vb = val_v[buf, r, pl.ds(NFIELDS - NLANE, NLANE)]
            for f in range(NFIELDS):
                s = va[f] if f < NLANE else vb[f - (NFIELDS - NLANE)]
                rows_v[buf, r, f, :] = rows_v[buf, r, f, :] * s
            return carry

        lax.fori_loop(0, RCHUNK, row_body, 0)

    gathers = [None, None]
    out_copies = [None, None]
    gathers[0] = stage(0, 0)
    for c in range(NCHUNK):
        buf = c & 1
        if c + 1 < NCHUNK:
            if out_copies[1 - buf] is not None:
                out_copies[1 - buf].wait()
                out_copies[1 - buf] = None
            gathers[1 - buf] = stage(c + 1, 1 - buf)
        gathers[buf].wait()
        scale(buf)
        out_copies[buf] = pltpu.async_copy(
            rows_v.at[buf],
            out_hbm.at[pl.ds(row0 + c * RCHUNK, RCHUNK), :, pl.ds(0, NEMB)],
            osem,
        )
    for cp in out_copies:
        if cp is not None:
            cp.wait()


TCOLS = 8192
TGRID = -(-NFEAT // TCOLS)


def _transpose_body(wt_ref, out_ref):
    z3 = wt_ref[...].T.reshape(TCOLS // 8, 8, NEMB)
    out_ref[...] = jnp.concatenate(
        [z3[:, j, :] for j in range(8)], axis=1
    )


_w_to_rows = pl.pallas_call(
    _transpose_body,
    grid=(TGRID,),
    in_specs=[pl.BlockSpec((NEMB, TCOLS), lambda g: (0, g))],
    out_specs=pl.BlockSpec((TCOLS // 8, 128), lambda g: (g, 0)),
    out_shape=jax.ShapeDtypeStruct((NFEAT // 8, 128), jnp.float32),
)


def kernel(id, value, W):
    w_rows = _w_to_rows(W.T).reshape(NFEAT, NEMB)
    out4 = _emb_lookup(w_rows, id.astype(jnp.int32), value)
    return out4[:, :NFIELDS, :NEMB]
